# baseline (speedup 1.0000x reference)
import jax
import jax.numpy as jnp
from jax.experimental import pallas as pl

_NUM_EXPERTS = 8
_TOP_K = 2
_EPSILON = 1.5
_NEG_FILL = -1000000000.0
_BLOCK_ROWS = 2048


def _router_block(x_ref, wg_ref, sp_ref, rw_ref, se_ref):
    x = x_ref[...]
    wg = wg_ref[...]
    sp = sp_ref[...]
    raw = jax.lax.dot_general(
        wg, x, (((1,), (1,)), ((), ())), preferred_element_type=jnp.float32)
    wm = jax.lax.dot_general(
        sp, x, (((0,), (1,)), ((), ())), preferred_element_type=jnp.float32)

    mx = jnp.max(raw, axis=0, keepdims=True)
    mod = jnp.where(raw >= mx - _EPSILON, wm, _NEG_FILL)

    iota = jax.lax.broadcasted_iota(jnp.int32, mod.shape, 0)
    m1 = jnp.max(mod, axis=0, keepdims=True)
    i1 = jnp.min(jnp.where(mod == m1, iota, _NUM_EXPERTS), axis=0, keepdims=True)
    mod2 = jnp.where(iota == i1, -jnp.inf, mod)
    m2 = jnp.max(mod2, axis=0, keepdims=True)
    i2 = jnp.min(jnp.where(mod2 == m2, iota, _NUM_EXPERTS), axis=0, keepdims=True)

    r1 = jnp.sum(jnp.where(iota == i1, raw, 0.0), axis=0, keepdims=True)
    r2 = jnp.sum(jnp.where(iota == i2, raw, 0.0), axis=0, keepdims=True)
    a = jnp.maximum(r1, r2)
    e1 = jnp.exp(r1 - a)
    e2 = jnp.exp(r2 - a)
    s = e1 + e2

    rw_t = jnp.concatenate([e1 / s, e2 / s], axis=0)
    se_t = jnp.concatenate([i1, i2], axis=0)
    rw_ref[...] = rw_t.T
    se_ref[...] = se_t.T


def kernel(hidden_states, W_gate, secret_projection):
    b, s, d = hidden_states.shape
    n = b * s
    x = hidden_states.reshape(n, d)

    grid = (n // _BLOCK_ROWS,)
    rw, se = pl.pallas_call(
        _router_block,
        grid=grid,
        in_specs=[
            pl.BlockSpec((_BLOCK_ROWS, d), lambda i: (i, 0)),
            pl.BlockSpec((_NUM_EXPERTS, d), lambda i: (0, 0)),
            pl.BlockSpec((d, _NUM_EXPERTS), lambda i: (0, 0)),
        ],
        out_specs=[
            pl.BlockSpec((_BLOCK_ROWS, _TOP_K), lambda i: (i, 0)),
            pl.BlockSpec((_BLOCK_ROWS, _TOP_K), lambda i: (i, 0)),
        ],
        out_shape=[
            jax.ShapeDtypeStruct((n, _TOP_K), jnp.float32),
            jax.ShapeDtypeStruct((n, _TOP_K), jnp.int32),
        ],
    )(x, W_gate, secret_projection)
    return rw.reshape(b, s, _TOP_K), se.reshape(b, s, _TOP_K)


# speedup vs baseline: 1.6841x; 1.6841x over previous
import jax
import jax.numpy as jnp
from jax.experimental import pallas as pl

_NUM_EXPERTS = 8
_TOP_K = 2
_EPSILON = 1.5
_NEG_FILL = -1000000000.0
_BLOCK_ROWS = 2048


def _router_block(x_ref, wg_ref, sp_ref, rw_ref, se_ref):
    x = x_ref[...]
    wg = wg_ref[...]
    sp = sp_ref[...]
    raw = jax.lax.dot_general(
        wg, x, (((1,), (1,)), ((), ())), preferred_element_type=jnp.float32)
    wm = jax.lax.dot_general(
        sp, x, (((0,), (1,)), ((), ())), preferred_element_type=jnp.float32)

    mx = jnp.max(raw, axis=0, keepdims=True)
    mod = jnp.where(raw >= mx - _EPSILON, wm, _NEG_FILL)

    iota = jax.lax.broadcasted_iota(jnp.int32, mod.shape, 0)
    m1 = jnp.max(mod, axis=0, keepdims=True)
    i1 = jnp.min(jnp.where(mod == m1, iota, _NUM_EXPERTS), axis=0, keepdims=True)
    mod2 = jnp.where(iota == i1, -jnp.inf, mod)
    m2 = jnp.max(mod2, axis=0, keepdims=True)
    i2 = jnp.min(jnp.where(mod2 == m2, iota, _NUM_EXPERTS), axis=0, keepdims=True)

    r1 = jnp.sum(jnp.where(iota == i1, raw, 0.0), axis=0, keepdims=True)
    r2 = jnp.sum(jnp.where(iota == i2, raw, 0.0), axis=0, keepdims=True)
    a = jnp.maximum(r1, r2)
    e1 = jnp.exp(r1 - a)
    e2 = jnp.exp(r2 - a)
    s = e1 + e2

    rw_ref[...] = jnp.concatenate([e1 / s, e2 / s], axis=0)
    se_ref[...] = jnp.concatenate([i1, i2], axis=0)


def kernel(hidden_states, W_gate, secret_projection):
    b, s, d = hidden_states.shape
    n = b * s
    x = hidden_states.reshape(n, d)

    grid = (n // _BLOCK_ROWS,)
    rw, se = pl.pallas_call(
        _router_block,
        grid=grid,
        in_specs=[
            pl.BlockSpec((_BLOCK_ROWS, d), lambda i: (i, 0)),
            pl.BlockSpec((_NUM_EXPERTS, d), lambda i: (0, 0)),
            pl.BlockSpec((d, _NUM_EXPERTS), lambda i: (0, 0)),
        ],
        out_specs=[
            pl.BlockSpec((_TOP_K, _BLOCK_ROWS), lambda i: (0, i)),
            pl.BlockSpec((_TOP_K, _BLOCK_ROWS), lambda i: (0, i)),
        ],
        out_shape=[
            jax.ShapeDtypeStruct((_TOP_K, n), jnp.float32),
            jax.ShapeDtypeStruct((_TOP_K, n), jnp.int32),
        ],
    )(x, W_gate, secret_projection)
    return rw.T.reshape(b, s, _TOP_K), se.T.reshape(b, s, _TOP_K)


# speedup vs baseline: 1.9734x; 1.1718x over previous
import jax
import jax.numpy as jnp
from jax.experimental import pallas as pl

_NUM_EXPERTS = 8
_TOP_K = 2
_EPSILON = 1.5
_NEG_FILL = -1000000000.0
_BLOCK_ROWS = 2048


def _router_block(x_ref, wt_ref, rw_ref, se_ref):
    x = x_ref[...]
    wt = wt_ref[...]
    logits_t = jax.lax.dot_general(
        wt, x, (((1,), (1,)), ((), ())), preferred_element_type=jnp.float32)
    raw = logits_t[:_NUM_EXPERTS, :]
    wm = logits_t[_NUM_EXPERTS:, :]

    mx = jnp.max(raw, axis=0, keepdims=True)
    mod = jnp.where(raw >= mx - _EPSILON, wm, _NEG_FILL)

    iota = jax.lax.broadcasted_iota(jnp.int32, mod.shape, 0)
    m1 = jnp.max(mod, axis=0, keepdims=True)
    i1 = jnp.min(jnp.where(mod == m1, iota, _NUM_EXPERTS), axis=0, keepdims=True)
    mod2 = jnp.where(iota == i1, -jnp.inf, mod)
    m2 = jnp.max(mod2, axis=0, keepdims=True)
    i2 = jnp.min(jnp.where(mod2 == m2, iota, _NUM_EXPERTS), axis=0, keepdims=True)

    r1 = jnp.sum(jnp.where(iota == i1, raw, 0.0), axis=0, keepdims=True)
    r2 = jnp.sum(jnp.where(iota == i2, raw, 0.0), axis=0, keepdims=True)
    a = jnp.maximum(r1, r2)
    e1 = jnp.exp(r1 - a)
    e2 = jnp.exp(r2 - a)
    s = e1 + e2

    rw_ref[...] = jnp.concatenate([e1 / s, e2 / s], axis=0)
    se_ref[...] = jnp.concatenate([i1, i2], axis=0)


def kernel(hidden_states, W_gate, secret_projection):
    b, s, d = hidden_states.shape
    n = b * s
    x = hidden_states.reshape(n, d)
    wt = jnp.concatenate([W_gate, secret_projection.T], axis=0)

    grid = (n // _BLOCK_ROWS,)
    rw, se = pl.pallas_call(
        _router_block,
        grid=grid,
        in_specs=[
            pl.BlockSpec((_BLOCK_ROWS, d), lambda i: (i, 0)),
            pl.BlockSpec((2 * _NUM_EXPERTS, d), lambda i: (0, 0)),
        ],
        out_specs=[
            pl.BlockSpec((_TOP_K, _BLOCK_ROWS), lambda i: (0, i)),
            pl.BlockSpec((_TOP_K, _BLOCK_ROWS), lambda i: (0, i)),
        ],
        out_shape=[
            jax.ShapeDtypeStruct((_TOP_K, n), jnp.float32),
            jax.ShapeDtypeStruct((_TOP_K, n), jnp.int32),
        ],
    )(x, wt)
    return rw.T.reshape(b, s, _TOP_K), se.T.reshape(b, s, _TOP_K)


# speedup vs baseline: 1.9958x; 1.0114x over previous
import jax
import jax.numpy as jnp
from jax.experimental import pallas as pl

_NUM_EXPERTS = 8
_TOP_K = 2
_EPSILON = 1.5
_NEG_FILL = -1000000000.0
_BLOCK_ROWS = 4096


def _router_block(x_ref, wt_ref, rw_ref, se_ref):
    x = x_ref[...]
    wt = wt_ref[...]
    logits_t = jax.lax.dot_general(
        wt, x, (((1,), (1,)), ((), ())), preferred_element_type=jnp.float32)
    raw = logits_t[:_NUM_EXPERTS, :]
    wm = logits_t[_NUM_EXPERTS:, :]

    mx = jnp.max(raw, axis=0, keepdims=True)
    mod = jnp.where(raw >= mx - _EPSILON, wm, _NEG_FILL)

    iota = jax.lax.broadcasted_iota(jnp.int32, mod.shape, 0)
    m1 = jnp.max(mod, axis=0, keepdims=True)
    i1 = jnp.min(jnp.where(mod == m1, iota, _NUM_EXPERTS), axis=0, keepdims=True)
    mod2 = jnp.where(iota == i1, -jnp.inf, mod)
    m2 = jnp.max(mod2, axis=0, keepdims=True)
    i2 = jnp.min(jnp.where(mod2 == m2, iota, _NUM_EXPERTS), axis=0, keepdims=True)

    r1 = jnp.sum(jnp.where(iota == i1, raw, 0.0), axis=0, keepdims=True)
    r2 = jnp.sum(jnp.where(iota == i2, raw, 0.0), axis=0, keepdims=True)
    a = jnp.maximum(r1, r2)
    e1 = jnp.exp(r1 - a)
    e2 = jnp.exp(r2 - a)
    s = e1 + e2

    rw_ref[...] = jnp.concatenate([e1 / s, e2 / s], axis=0)
    se_ref[...] = jnp.concatenate([i1, i2], axis=0)


def kernel(hidden_states, W_gate, secret_projection):
    b, s, d = hidden_states.shape
    n = b * s
    x = hidden_states.reshape(n, d)
    wt = jnp.concatenate([W_gate, secret_projection.T], axis=0)

    grid = (n // _BLOCK_ROWS,)
    rw, se = pl.pallas_call(
        _router_block,
        grid=grid,
        in_specs=[
            pl.BlockSpec((_BLOCK_ROWS, d), lambda i: (i, 0)),
            pl.BlockSpec((2 * _NUM_EXPERTS, d), lambda i: (0, 0)),
        ],
        out_specs=[
            pl.BlockSpec((_TOP_K, _BLOCK_ROWS), lambda i: (0, i)),
            pl.BlockSpec((_TOP_K, _BLOCK_ROWS), lambda i: (0, i)),
        ],
        out_shape=[
            jax.ShapeDtypeStruct((_TOP_K, n), jnp.float32),
            jax.ShapeDtypeStruct((_TOP_K, n), jnp.int32),
        ],
    )(x, wt)
    return rw.T.reshape(b, s, _TOP_K), se.T.reshape(b, s, _TOP_K)
